# Initial kernel scaffold; baseline (speedup 1.0000x reference)
import jax
import jax.numpy as jnp
from jax.experimental import pallas as pl


def kernel(o_embedding, edge_weight, user_table, W1_0, b1_0, W2_0, b2_0, W1_1, b1_1, W2_1, b2_1, edge_index, u_id):
    raise NotImplementedError("write your pallas kernel here")


# speedup vs baseline: 4.1363x; 4.1363x over previous
import functools

import jax
import jax.numpy as jnp
from jax import lax
from jax.experimental import pallas as pl
from jax.experimental.pallas import tpu as pltpu
from jax.experimental.pallas import tpu_sc as plsc

N_NODES = 10000
D = 128
E_TOTAL = 320000
NC = 2
NS = 16
NW = NC * NS
EPW = E_TOTAL // NW
CHUNK = 80
NCHUNK = EPW // CHUNK
STRIPE = 624
TAIL = N_NODES - 15 * STRIPE


@functools.partial(
    pl.kernel,
    out_type=jax.ShapeDtypeStruct((NC, N_NODES, D), jnp.float32),
    mesh=plsc.VectorSubcoreMesh(core_axis_name="c", subcore_axis_name="s"),
    scratch_types=[
        pltpu.VMEM_SHARED((N_NODES, D), jnp.float32),
        pltpu.VMEM((CHUNK,), jnp.int32),
        pltpu.VMEM((CHUNK,), jnp.int32),
        pltpu.VMEM((CHUNK,), jnp.float32),
        pltpu.VMEM((CHUNK, D), jnp.float32),
        pltpu.SemaphoreType.DMA,
    ],
)
def _edge_pass(ego_hbm, src_hbm, dst_hbm, w_hbm, zeros_hbm, out_hbm,
               side_sh, src_v, dst_v, w_v, rows_v, sem):
    cid = lax.axis_index("c")
    sid = lax.axis_index("s")
    wid = sid * NC + cid
    base_row = sid * STRIPE

    pltpu.sync_copy(zeros_hbm.at[pl.ds(0, STRIPE)],
                    side_sh.at[pl.ds(base_row, STRIPE)])

    @pl.when(sid == NS - 1)
    def _zero_tail():
        pltpu.sync_copy(zeros_hbm.at[pl.ds(0, TAIL - STRIPE)],
                        side_sh.at[pl.ds(15 * STRIPE + STRIPE, TAIL - STRIPE)])

    plsc.subcore_barrier()

    def chunk_body(k, carry):
        base = wid * EPW + k * CHUNK
        pltpu.sync_copy(src_hbm.at[pl.ds(base, CHUNK)], src_v)
        pltpu.sync_copy(dst_hbm.at[pl.ds(base, CHUNK)], dst_v)
        pltpu.sync_copy(w_hbm.at[pl.ds(base, CHUNK)], w_v)
        pltpu.async_copy(ego_hbm.at[src_v], rows_v, sem).wait()

        def scale_body(e16, c2):
            wv = w_v[pl.ds(e16 * 16, 16)]
            for j in range(16):
                e = e16 * 16 + j
                we = wv[j]
                for g in range(D // 16):
                    sl = pl.ds(g * 16, 16)
                    rows_v[e, sl] = rows_v[e, sl] * we
            return c2

        lax.fori_loop(0, CHUNK // 16, scale_body, 0)
        pltpu.sync_copy(rows_v, side_sh.at[dst_v], add=True)
        return carry

    lax.fori_loop(0, NCHUNK, chunk_body, 0)
    plsc.subcore_barrier()

    pltpu.sync_copy(side_sh.at[pl.ds(base_row, STRIPE)],
                    out_hbm.at[cid, pl.ds(base_row, STRIPE)])

    @pl.when(sid == NS - 1)
    def _write_tail():
        pltpu.sync_copy(side_sh.at[pl.ds(16 * STRIPE, TAIL - STRIPE)],
                        out_hbm.at[cid, pl.ds(16 * STRIPE, TAIL - STRIPE)])


_BR = 1000


def _dense_body(side_ref, ego_ref, all_ref, w1_ref, b1_ref, w2_ref, b2_ref,
                ego_out_ref, all_out_ref):
    side = side_ref[0] + side_ref[1]
    ego = ego_ref[...]
    sum_e = jnp.dot(side, w1_ref[...], preferred_element_type=jnp.float32)
    bi = jnp.dot(ego * side, w2_ref[...], preferred_element_type=jnp.float32)
    h = sum_e + bi + b1_ref[...] + b2_ref[...]
    ego_o = jnp.where(h >= 0, h, 0.01 * h)
    nrm = jnp.maximum(
        jnp.sqrt(jnp.sum(ego_o * ego_o, axis=1, keepdims=True)), 1e-12)
    ego_out_ref[...] = ego_o
    all_out_ref[...] = all_ref[...] + ego_o / nrm


def _dense_pass(side_p, ego, all_emb, w1t, b1, w2t, b2):
    grid = (N_NODES // _BR,)
    return pl.pallas_call(
        _dense_body,
        grid=grid,
        in_specs=[
            pl.BlockSpec((NC, _BR, D), lambda i: (0, i, 0)),
            pl.BlockSpec((_BR, D), lambda i: (i, 0)),
            pl.BlockSpec((_BR, D), lambda i: (i, 0)),
            pl.BlockSpec((D, D), lambda i: (0, 0)),
            pl.BlockSpec((1, D), lambda i: (0, 0)),
            pl.BlockSpec((D, D), lambda i: (0, 0)),
            pl.BlockSpec((1, D), lambda i: (0, 0)),
        ],
        out_specs=[
            pl.BlockSpec((_BR, D), lambda i: (i, 0)),
            pl.BlockSpec((_BR, D), lambda i: (i, 0)),
        ],
        out_shape=[
            jax.ShapeDtypeStruct((N_NODES, D), jnp.float32),
            jax.ShapeDtypeStruct((N_NODES, D), jnp.float32),
        ],
    )(side_p, ego, all_emb, w1t, b1, w2t, b2)


def kernel(o_embedding, edge_weight, user_table, W1_0, b1_0, W2_0, b2_0,
           W1_1, b1_1, W2_1, b2_1, edge_index, u_id):
    u_emb = jnp.take(user_table, u_id, axis=0)
    ego = jnp.concatenate([u_emb, o_embedding], axis=0)
    all_emb = ego
    src = edge_index[0]
    dst = edge_index[1]
    params = [
        (W1_0.T, b1_0.reshape(1, D), W2_0.T, b2_0.reshape(1, D)),
        (W1_1.T, b1_1.reshape(1, D), W2_1.T, b2_1.reshape(1, D)),
    ]
    zeros = jnp.zeros((STRIPE, D), jnp.float32)
    for (w1t, b1, w2t, b2) in params:
        side_p = _edge_pass(ego, src, dst, edge_weight, zeros)
        ego, all_emb = _dense_pass(side_p, ego, all_emb, w1t, b1, w2t, b2)
    return all_emb
